# Initial kernel scaffold; baseline (speedup 1.0000x reference)
import jax
import jax.numpy as jnp
from jax.experimental import pallas as pl


def kernel(token_ids, weights):
    raise NotImplementedError("write your pallas kernel here")


# speedup vs baseline: 6.3415x; 6.3415x over previous
import functools

import jax
import jax.numpy as jnp
from jax import lax
from jax.experimental import pallas as pl
from jax.experimental.pallas import tpu as pltpu
from jax.experimental.pallas import tpu_sc as plsc

D_MODEL = 128
NUM_WORKERS = 32
CHUNK = 128


def _emb_body(idx_hbm, table_hbm, out_hbm, idx_v, rows_v, gsem):
    nc = 2
    wid = lax.axis_index("s") * nc + lax.axis_index("c")
    pltpu.sync_copy(idx_hbm.at[wid], idx_v)
    n_chunks = idx_v.shape[0]
    base = wid * n_chunks * CHUNK

    def body(j, carry):
        pltpu.async_copy(table_hbm.at[idx_v.at[j]], rows_v, gsem).wait()
        pltpu.sync_copy(rows_v, out_hbm.at[pl.ds(base + j * CHUNK, CHUNK)])
        return carry

    lax.fori_loop(0, n_chunks, body, 0)


@functools.partial(jax.jit, static_argnums=())
def _emb_call(idx3, weights):
    n_chunks = idx3.shape[1]
    total = NUM_WORKERS * n_chunks * CHUNK
    mesh = plsc.VectorSubcoreMesh(core_axis_name="c", subcore_axis_name="s")
    f = pl.kernel(
        _emb_body,
        mesh=mesh,
        out_type=jax.ShapeDtypeStruct((total, D_MODEL), jnp.float32),
        scratch_types=[
            pltpu.VMEM((n_chunks, CHUNK), jnp.int32),
            pltpu.VMEM((CHUNK, D_MODEL), jnp.float32),
            pltpu.SemaphoreType.DMA,
        ],
    )
    return f(idx3, weights)


def kernel(token_ids, weights):
    b, s = token_ids.shape
    idx3 = token_ids.astype(jnp.int32).reshape(NUM_WORKERS, -1, CHUNK)
    out = _emb_call(idx3, weights)
    return out.reshape(b, s, D_MODEL)


# speedup vs baseline: 9.1911x; 1.4494x over previous
import functools

import jax
import jax.numpy as jnp
from jax import lax
from jax.experimental import pallas as pl
from jax.experimental.pallas import tpu as pltpu
from jax.experimental.pallas import tpu_sc as plsc

D_MODEL = 128
NUM_WORKERS = 32
CHUNK = 128
NBUF = 4


def _emb_body(idx_hbm, table_hbm, out_hbm, idx_v, rows_v,
              gs0, gs1, gs2, gs3, ws0, ws1, ws2, ws3):
    gsems = (gs0, gs1, gs2, gs3)
    wsems = (ws0, ws1, ws2, ws3)
    nc = 2
    wid = lax.axis_index("s") * nc + lax.axis_index("c")
    pltpu.sync_copy(idx_hbm.at[wid], idx_v)
    n_chunks = idx_v.shape[0]
    base = wid * n_chunks * CHUNK

    def start_g(jj, b):
        pltpu.async_copy(table_hbm.at[idx_v.at[jj]], rows_v.at[b], gsems[b])

    def wait_g(b):
        pltpu.make_async_copy(table_hbm.at[idx_v.at[0]], rows_v.at[b],
                              gsems[b]).wait()

    def start_w(jj, b):
        pltpu.async_copy(rows_v.at[b],
                         out_hbm.at[pl.ds(base + jj * CHUNK, CHUNK)], wsems[b])

    def wait_w(b):
        pltpu.make_async_copy(rows_v.at[b], out_hbm.at[pl.ds(0, CHUNK)],
                              wsems[b]).wait()

    start_g(0, 0)
    start_g(1, 1)
    wait_g(0)
    start_w(0, 0)
    start_g(2, 2)
    wait_g(1)
    start_w(1, 1)
    start_g(3, 3)

    def body(i, carry):
        j0 = 2 + i * NBUF
        for off in range(NBUF):
            jj = j0 + off
            b = (2 + off) % NBUF
            wait_g(b)
            start_w(jj, b)
            b2 = (b + 2) % NBUF
            wait_w(b2)
            start_g(jj + 2, b2)
        return carry

    lax.fori_loop(0, (n_chunks - 4) // NBUF, body, 0)

    for jj in (n_chunks - 2, n_chunks - 1):
        b = jj % NBUF
        wait_g(b)
        start_w(jj, b)
        wait_w((b + 2) % NBUF)
    wait_w((n_chunks - 2) % NBUF)
    wait_w((n_chunks - 1) % NBUF)


@functools.partial(jax.jit, static_argnums=())
def _emb_call(idx3, weights):
    n_chunks = idx3.shape[1]
    total = NUM_WORKERS * n_chunks * CHUNK
    mesh = plsc.VectorSubcoreMesh(core_axis_name="c", subcore_axis_name="s")
    f = pl.kernel(
        _emb_body,
        mesh=mesh,
        out_type=jax.ShapeDtypeStruct((total, D_MODEL), jnp.float32),
        scratch_types=[
            pltpu.VMEM((n_chunks, CHUNK), jnp.int32),
            pltpu.VMEM((NBUF, CHUNK, D_MODEL), jnp.float32),
        ] + [pltpu.SemaphoreType.DMA] * (2 * NBUF),
    )
    return f(idx3, weights)


def kernel(token_ids, weights):
    b, s = token_ids.shape
    idx3 = token_ids.astype(jnp.int32).reshape(NUM_WORKERS, -1, CHUNK)
    out = _emb_call(idx3, weights)
    return out.reshape(b, s, D_MODEL)


# speedup vs baseline: 9.2333x; 1.0046x over previous
import functools

import jax
import jax.numpy as jnp
from jax import lax
from jax.experimental import pallas as pl
from jax.experimental.pallas import tpu as pltpu
from jax.experimental.pallas import tpu_sc as plsc

D_MODEL = 128
NUM_WORKERS = 32
CHUNK = 128
NBUF = 6
LOOKAHEAD = NBUF // 2


def _emb_body(idx_hbm, table_hbm, out_hbm, idx_v, rows_v, *sems):
    gsems = sems[:NBUF]
    wsems = sems[NBUF:]
    nc = 2
    wid = lax.axis_index("s") * nc + lax.axis_index("c")
    pltpu.sync_copy(idx_hbm.at[wid], idx_v)
    n = idx_v.shape[0]
    base = wid * n * CHUNK
    L = LOOKAHEAD

    def start_g(jj, b):
        pltpu.async_copy(table_hbm.at[idx_v.at[jj]], rows_v.at[b], gsems[b])

    def wait_g(b):
        pltpu.make_async_copy(table_hbm.at[idx_v.at[0]], rows_v.at[b],
                              gsems[b]).wait()

    def start_w(jj, b):
        pltpu.async_copy(rows_v.at[b],
                         out_hbm.at[pl.ds(base + jj * CHUNK, CHUNK)], wsems[b])

    def wait_w(b):
        pltpu.make_async_copy(rows_v.at[b], out_hbm.at[pl.ds(0, CHUNK)],
                              wsems[b]).wait()

    jj0 = NBUF - L
    m = (n - jj0 - L) // NBUF
    tail = jj0 + m * NBUF

    for jj in range(L):
        start_g(jj, jj % NBUF)
    for jj in range(jj0):
        wait_g(jj % NBUF)
        start_w(jj, jj % NBUF)
        start_g(jj + L, (jj + L) % NBUF)

    def body(i, carry):
        j0 = jj0 + i * NBUF
        for off in range(NBUF):
            jj = j0 + off
            b = (jj0 + off) % NBUF
            wait_g(b)
            start_w(jj, b)
            b2 = (b + L) % NBUF
            wait_w(b2)
            start_g(jj + L, b2)
        return carry

    lax.fori_loop(0, m, body, 0)

    for jj in range(tail, n):
        b = jj % NBUF
        wait_g(b)
        start_w(jj, b)
        wait_w((b + L) % NBUF)
        if jj + L < n:
            start_g(jj + L, (jj + L) % NBUF)
    for jj in range(n - NBUF + L, n):
        wait_w(jj % NBUF)


@functools.partial(jax.jit, static_argnums=())
def _emb_call(idx3, weights):
    n_chunks = idx3.shape[1]
    total = NUM_WORKERS * n_chunks * CHUNK
    mesh = plsc.VectorSubcoreMesh(core_axis_name="c", subcore_axis_name="s")
    f = pl.kernel(
        _emb_body,
        mesh=mesh,
        out_type=jax.ShapeDtypeStruct((total, D_MODEL), jnp.float32),
        scratch_types=[
            pltpu.VMEM((n_chunks, CHUNK), jnp.int32),
            pltpu.VMEM((NBUF, CHUNK, D_MODEL), jnp.float32),
        ] + [pltpu.SemaphoreType.DMA] * (2 * NBUF),
    )
    return f(idx3, weights)


def kernel(token_ids, weights):
    b, s = token_ids.shape
    idx3 = token_ids.astype(jnp.int32).reshape(NUM_WORKERS, -1, CHUNK)
    out = _emb_call(idx3, weights)
    return out.reshape(b, s, D_MODEL)
